# baseline (speedup 1.0000x reference)
import functools

import jax
import jax.numpy as jnp
from jax import lax
from jax.experimental import pallas as pl
from jax.experimental.pallas import tpu as pltpu
from jax.experimental.pallas import tpu_sc as plsc


_NSEG = 16
_L = 2048
_BI = 256
_CH = 512
_NCH = _L // _CH
_LANES = 16


def _sc_partition_kernel(x_hbm, lab_hbm, xs_hbm, cnt_hbm, lab_v, x_v, out_v,
                         cnt_v):
    core = lax.axis_index("c")
    sub = lax.axis_index("s")

    @pl.when(core == 0)
    def _body():
        seg = sub
        pltpu.sync_copy(lab_hbm.at[seg], lab_v)
        pltpu.sync_copy(x_hbm.at[seg], x_v)

        i32 = jnp.int32
        nchunks = _L // _LANES
        zv = jnp.zeros((_LANES,), i32)
        onev = jnp.full((_LANES,), 1, i32)

        def hist_body(i, carry):
            a0, a1, a2, a3 = carry
            lv = lab_v[pl.ds(i * _LANES, _LANES)]
            a0 = a0 + plsc.all_reduce_population_count(lv == 0)
            a1 = a1 + plsc.all_reduce_population_count(lv == 1)
            a2 = a2 + plsc.all_reduce_population_count(lv == 2)
            a3 = a3 + plsc.all_reduce_population_count(lv == 3)
            return a0, a1, a2, a3

        c0v, c1v, c2v, c3v = lax.fori_loop(0, nchunks, hist_body,
                                           (zv, zv, zv, zv))
        o1v = c0v
        o2v = c0v + c1v
        o3v = o2v + c2v
        o4v = o3v + c3v

        def part_body(i, carry):
            r0, r1, r2, r3, r4 = carry
            lv = lab_v[pl.ds(i * _LANES, _LANES)]
            xv = x_v[pl.ds(i * _LANES, _LANES)]
            outs = []
            for a, r in ((0, r0), (1, r1), (2, r2), (3, r3), (4, r4)):
                m = lv == a
                mi = jnp.where(m, onev, zv)
                pos = plsc.cumsum(mi)
                idx = r + pos - onev
                plsc.store_scatter(out_v, [idx], xv, mask=m)
                cnt = plsc.all_reduce_population_count(m)
                outs.append(r + cnt)
            return tuple(outs)

        init = (zv, o1v, o2v, o3v, o4v)
        lax.fori_loop(0, nchunks, part_body, init)

        io = lax.iota(i32, _LANES)
        cvec = (jnp.where(io == 0, c0v, zv) + jnp.where(io == 1, c1v, zv)
                + jnp.where(io == 2, c2v, zv) + jnp.where(io == 3, c3v, zv))
        cnt_v[...] = cvec
        pltpu.sync_copy(out_v, xs_hbm.at[seg])
        pltpu.sync_copy(cnt_v, cnt_hbm.at[seg])


@functools.cache
def _get_sc_partition():
    return pl.kernel(
        _sc_partition_kernel,
        out_type=(
            jax.ShapeDtypeStruct((_NSEG, _L), jnp.float32),
            jax.ShapeDtypeStruct((_NSEG, _LANES), jnp.int32),
        ),
        mesh=plsc.VectorSubcoreMesh(core_axis_name="c", subcore_axis_name="s"),
        compiler_params=pltpu.CompilerParams(needs_layout_passes=False),
        scratch_types=[
            pltpu.VMEM((_L,), jnp.int32),
            pltpu.VMEM((_L,), jnp.float32),
            pltpu.VMEM((_L,), jnp.float32),
            pltpu.VMEM((_LANES,), jnp.int32),
        ],
    )


def _tc_kernel(xs_ch_ref, xs_col_ref, cnt_ref, out_ref, sf_ref, si_ref):
    s = pl.program_id(0)
    ib = pl.program_id(1)

    @pl.when(jnp.logical_and(s == 0, ib == 0))
    def _init():
        out_ref[...] = jnp.zeros((1, 1), jnp.float32)

    f32 = jnp.float32
    zero = jnp.float32(0.0)
    one = jnp.float32(1.0)

    @pl.when(ib == 0)
    def _per_segment():
        cnts = cnt_ref[0, 0]
        c0 = cnts[0]
        c1 = cnts[1]
        c2 = cnts[2]
        c3 = cnts[3]
        o1 = c0
        o2 = c0 + c1
        o3 = o2 + c2
        o4 = o3 + c3
        c1f = c1.astype(f32)
        c2f = c2.astype(f32)
        c3f = c3.astype(f32)
        n1f = o1.astype(f32)
        n2f = o2.astype(f32)
        n3f = o3.astype(f32)
        inc1 = jnp.logical_and(c1 > 0, o1 > 0).astype(f32)
        inc2 = jnp.logical_and(c2 > 0, o2 > 0).astype(f32)
        inc3 = jnp.logical_and(c3 > 0, o3 > 0).astype(f32)
        si_ref[0] = o1
        si_ref[1] = o2
        si_ref[2] = o3
        si_ref[3] = o4
        sf_ref[0] = inc1 / jnp.maximum(c1f * n1f, 1.0)
        sf_ref[1] = inc2 / jnp.maximum(c2f * n2f, 1.0)
        sf_ref[2] = inc3 / jnp.maximum(c3f * n3f, 1.0)
        sf_ref[3] = jnp.maximum(inc1 + inc2 + inc3, 1.0)
        sf_ref[4] = inc1 / jnp.maximum(c1f, 1.0)
        sf_ref[5] = inc2 / jnp.maximum(c2f, 1.0)
        sf_ref[6] = inc3 / jnp.maximum(c3f, 1.0)
        sf_ref[7] = jnp.max(xs_ch_ref[0])

    o1 = si_ref[0]
    o2 = si_ref[1]
    o3 = si_ref[2]
    o4 = si_ref[3]
    r1 = sf_ref[0]
    r2 = sf_ref[1]
    r3 = sf_ref[2]
    term_cnt = sf_ref[3]
    v1 = sf_ref[4]
    v2 = sf_ref[5]
    v3 = sf_ref[6]
    m = sf_ref[7]

    gi = ib * _BI + lax.broadcasted_iota(jnp.int32, (_BI, 1), 0)
    zi = jnp.zeros((_BI, 1), jnp.int32)
    zf = jnp.zeros((_BI, 1), f32)
    in1 = gi < o2
    in2 = gi < o3
    in3 = gi < o4
    in0 = gi < o1
    b_i = jnp.where(in0, zi,
                    jnp.where(in1, o1, jnp.where(in2, o2,
                                                 jnp.where(in3, o3, zi))))
    w_i = jnp.where(in0, zf,
                    jnp.where(in1, r1, jnp.where(in2, r2,
                                                 jnp.where(in3, r3, zf))))
    wl_i = jnp.where(in0, zf,
                     jnp.where(in1, v1, jnp.where(in2, v2,
                                                  jnp.where(in3, v3, zf))))

    xi = xs_col_ref[0]
    xim = xi - m
    ei = jnp.exp(xim)
    lin_b = jnp.sum(wl_i * xim)

    bmax = jnp.max(b_i)
    nch = (bmax + (_CH - 1)) // _CH
    jio = lax.broadcasted_iota(jnp.int32, (1, _CH), 1)

    def chunk_body(cix, acc):
        xch = xs_ch_ref[0, cix]
        ec = jnp.exp(xch - m)
        mask = (jio + cix * _CH) < b_i
        return acc + jnp.log(jnp.where(mask, ei + ec, one))

    acc = jnp.zeros((_BI, _CH), f32)
    acc = lax.fori_loop(0, nch, chunk_body, acc)
    pair_log = jnp.sum(acc * w_i)

    nseg = pl.num_programs(0)
    upd = -(lin_b - pair_log) / (term_cnt * nseg)
    out_ref[...] += jnp.broadcast_to(upd, (1, 1))


def _tc_loss(xs, cnt):
    xs_ch = xs.reshape(_NSEG, _NCH, 1, _CH)
    xs_col = xs.reshape(_NSEG, _L, 1)
    cnt3 = cnt.reshape(_NSEG, 1, _LANES)
    out = pl.pallas_call(
        _tc_kernel,
        grid=(_NSEG, _L // _BI),
        in_specs=[
            pl.BlockSpec((1, _NCH, 1, _CH), lambda s, ib: (s, 0, 0, 0)),
            pl.BlockSpec((1, _BI, 1), lambda s, ib: (s, ib, 0)),
            pl.BlockSpec((1, 1, _LANES), lambda s, ib: (s, 0, 0)),
        ],
        out_specs=pl.BlockSpec((1, 1), lambda s, ib: (0, 0)),
        out_shape=jax.ShapeDtypeStruct((1, 1), jnp.float32),
        scratch_shapes=[
            pltpu.SMEM((8,), jnp.float32),
            pltpu.SMEM((8,), jnp.int32),
        ],
    )(xs_ch, xs_col, cnt3)
    return out[0, 0]


@jax.jit
def _bpr_loss(logits, labels):
    x2d = logits.reshape(_NSEG, _L)
    lab2d = labels.reshape(_NSEG, _L)
    xs, cnt = _get_sc_partition()(x2d, lab2d)
    return _tc_loss(xs, cnt)


def kernel(s_num, logits, labels):
    return _bpr_loss(logits, labels)


# speedup vs baseline: 1.0321x; 1.0321x over previous
import functools

import jax
import jax.numpy as jnp
from jax import lax
from jax.experimental import pallas as pl
from jax.experimental.pallas import tpu as pltpu
from jax.experimental.pallas import tpu_sc as plsc


_NSEG = 16
_L = 2048
_BI = 256
_CH = 512
_NCH = _L // _CH
_LANES = 16


def _sc_partition_kernel(x_hbm, lab_hbm, xs_hbm, cnt_hbm, lab_v, x_v, out_v,
                         cnt_v):
    core = lax.axis_index("c")
    sub = lax.axis_index("s")

    @pl.when(core == 0)
    def _body():
        seg = sub
        pltpu.sync_copy(lab_hbm.at[seg], lab_v)
        pltpu.sync_copy(x_hbm.at[seg], x_v)

        i32 = jnp.int32
        nchunks = _L // _LANES
        zv = jnp.zeros((_LANES,), i32)
        onev = jnp.full((_LANES,), 1, i32)

        def hist_body(i, carry):
            a0, a1, a2, a3 = carry
            lv = lab_v[pl.ds(i * _LANES, _LANES)]
            a0 = a0 + plsc.all_reduce_population_count(lv == 0)
            a1 = a1 + plsc.all_reduce_population_count(lv == 1)
            a2 = a2 + plsc.all_reduce_population_count(lv == 2)
            a3 = a3 + plsc.all_reduce_population_count(lv == 3)
            return a0, a1, a2, a3

        c0v, c1v, c2v, c3v = lax.fori_loop(0, nchunks, hist_body,
                                           (zv, zv, zv, zv))
        o1v = c0v
        o2v = c0v + c1v
        o3v = o2v + c2v
        o4v = o3v + c3v

        def part_body(i, carry):
            r0, r1, r2, r3, r4 = carry
            lv = lab_v[pl.ds(i * _LANES, _LANES)]
            xv = x_v[pl.ds(i * _LANES, _LANES)]
            outs = []
            for a, r in ((0, r0), (1, r1), (2, r2), (3, r3), (4, r4)):
                m = lv == a
                mi = jnp.where(m, onev, zv)
                pos = plsc.cumsum(mi)
                idx = r + pos - onev
                plsc.store_scatter(out_v, [idx], xv, mask=m)
                cnt = plsc.all_reduce_population_count(m)
                outs.append(r + cnt)
            return tuple(outs)

        init = (zv, o1v, o2v, o3v, o4v)
        lax.fori_loop(0, nchunks, part_body, init)

        io = lax.iota(i32, _LANES)
        cvec = (jnp.where(io == 0, c0v, zv) + jnp.where(io == 1, c1v, zv)
                + jnp.where(io == 2, c2v, zv) + jnp.where(io == 3, c3v, zv))
        cnt_v[...] = cvec
        pltpu.sync_copy(out_v, xs_hbm.at[seg])
        pltpu.sync_copy(cnt_v, cnt_hbm.at[seg])


@functools.cache
def _get_sc_partition():
    return pl.kernel(
        _sc_partition_kernel,
        out_type=(
            jax.ShapeDtypeStruct((_NSEG, _L), jnp.float32),
            jax.ShapeDtypeStruct((_NSEG, _LANES), jnp.int32),
        ),
        mesh=plsc.VectorSubcoreMesh(core_axis_name="c", subcore_axis_name="s"),
        compiler_params=pltpu.CompilerParams(needs_layout_passes=False),
        scratch_types=[
            pltpu.VMEM((_L,), jnp.int32),
            pltpu.VMEM((_L,), jnp.float32),
            pltpu.VMEM((_L,), jnp.float32),
            pltpu.VMEM((_LANES,), jnp.int32),
        ],
    )


def _tc_kernel(xs_ch_ref, xs_col_ref, cnt_ref, out_ref, sf_ref, si_ref):
    s = pl.program_id(0)
    ib = pl.program_id(1)

    @pl.when(jnp.logical_and(s == 0, ib == 0))
    def _init():
        out_ref[...] = jnp.zeros((1, 1), jnp.float32)

    f32 = jnp.float32
    zero = jnp.float32(0.0)
    one = jnp.float32(1.0)

    @pl.when(ib == 0)
    def _per_segment():
        cnts = cnt_ref[0, 0]
        c0 = cnts[0]
        c1 = cnts[1]
        c2 = cnts[2]
        c3 = cnts[3]
        o1 = c0
        o2 = c0 + c1
        o3 = o2 + c2
        o4 = o3 + c3
        c1f = c1.astype(f32)
        c2f = c2.astype(f32)
        c3f = c3.astype(f32)
        n1f = o1.astype(f32)
        n2f = o2.astype(f32)
        n3f = o3.astype(f32)
        inc1 = jnp.logical_and(c1 > 0, o1 > 0).astype(f32)
        inc2 = jnp.logical_and(c2 > 0, o2 > 0).astype(f32)
        inc3 = jnp.logical_and(c3 > 0, o3 > 0).astype(f32)
        si_ref[0] = o1
        si_ref[1] = o2
        si_ref[2] = o3
        si_ref[3] = o4
        sf_ref[0] = inc1 / jnp.maximum(c1f * n1f, 1.0)
        sf_ref[1] = inc2 / jnp.maximum(c2f * n2f, 1.0)
        sf_ref[2] = inc3 / jnp.maximum(c3f * n3f, 1.0)
        sf_ref[3] = jnp.maximum(inc1 + inc2 + inc3, 1.0)
        sf_ref[4] = inc1 / jnp.maximum(c1f, 1.0)
        sf_ref[5] = inc2 / jnp.maximum(c2f, 1.0)
        sf_ref[6] = inc3 / jnp.maximum(c3f, 1.0)
        sf_ref[7] = jnp.max(xs_ch_ref[0])

    o1 = si_ref[0]
    o2 = si_ref[1]
    o3 = si_ref[2]
    o4 = si_ref[3]
    r1 = sf_ref[0]
    r2 = sf_ref[1]
    r3 = sf_ref[2]
    term_cnt = sf_ref[3]
    v1 = sf_ref[4]
    v2 = sf_ref[5]
    v3 = sf_ref[6]
    m = sf_ref[7]

    gi = ib * _BI + lax.broadcasted_iota(jnp.int32, (_BI, 1), 0)
    zi = jnp.zeros((_BI, 1), jnp.int32)
    zf = jnp.zeros((_BI, 1), f32)
    in1 = gi < o2
    in2 = gi < o3
    in3 = gi < o4
    in0 = gi < o1
    b_i = jnp.where(in0, zi,
                    jnp.where(in1, o1, jnp.where(in2, o2,
                                                 jnp.where(in3, o3, zi))))
    w_i = jnp.where(in0, zf,
                    jnp.where(in1, r1, jnp.where(in2, r2,
                                                 jnp.where(in3, r3, zf))))
    wl_i = jnp.where(in0, zf,
                     jnp.where(in1, v1, jnp.where(in2, v2,
                                                  jnp.where(in3, v3, zf))))

    xi = xs_col_ref[0]
    xim = xi - m
    ei = jnp.exp(xim)
    lin_b = jnp.sum(wl_i * xim)

    bmax = jnp.max(b_i)
    nch = (bmax + (_CH - 1)) // _CH
    jio = lax.broadcasted_iota(jnp.int32, (1, _CH), 1)

    def chunk_body(cix, acc):
        xch = xs_ch_ref[0, cix]
        ec = jnp.exp(xch - m)
        mask = (jio + cix * _CH) < b_i
        t = jnp.log(jnp.where(mask, ei + ec, one))
        return acc + jnp.sum(t * w_i)

    pair_log = lax.fori_loop(0, nch, chunk_body, jnp.float32(0.0))

    nseg = pl.num_programs(0)
    upd = -(lin_b - pair_log) / (term_cnt * nseg)
    out_ref[...] += jnp.broadcast_to(upd, (1, 1))


def _tc_loss(xs, cnt):
    xs_ch = xs.reshape(_NSEG, _NCH, 1, _CH)
    xs_col = xs.reshape(_NSEG, _L, 1)
    cnt3 = cnt.reshape(_NSEG, 1, _LANES)
    out = pl.pallas_call(
        _tc_kernel,
        grid=(_NSEG, _L // _BI),
        in_specs=[
            pl.BlockSpec((1, _NCH, 1, _CH), lambda s, ib: (s, 0, 0, 0)),
            pl.BlockSpec((1, _BI, 1), lambda s, ib: (s, ib, 0)),
            pl.BlockSpec((1, 1, _LANES), lambda s, ib: (s, 0, 0)),
        ],
        out_specs=pl.BlockSpec((1, 1), lambda s, ib: (0, 0)),
        out_shape=jax.ShapeDtypeStruct((1, 1), jnp.float32),
        scratch_shapes=[
            pltpu.SMEM((8,), jnp.float32),
            pltpu.SMEM((8,), jnp.int32),
        ],
    )(xs_ch, xs_col, cnt3)
    return out[0, 0]


@jax.jit
def _bpr_loss(logits, labels):
    x2d = logits.reshape(_NSEG, _L)
    lab2d = labels.reshape(_NSEG, _L)
    xs, cnt = _get_sc_partition()(x2d, lab2d)
    return _tc_loss(xs, cnt)


def kernel(s_num, logits, labels):
    return _bpr_loss(logits, labels)


# speedup vs baseline: 1.0325x; 1.0004x over previous
import functools

import jax
import jax.numpy as jnp
from jax import lax
from jax.experimental import pallas as pl
from jax.experimental.pallas import tpu as pltpu
from jax.experimental.pallas import tpu_sc as plsc


_NSEG = 16
_L = 2048
_BI = 256
_CH = 512
_NCH = _L // _CH
_LANES = 16


def _sc_partition_kernel(x_hbm, lab_hbm, xs_hbm, cnt_hbm, lab_v, x_v, out_v,
                         cnt_v):
    core = lax.axis_index("c")
    sub = lax.axis_index("s")

    @pl.when(core == 0)
    def _body():
        seg = sub
        pltpu.sync_copy(lab_hbm.at[seg], lab_v)
        pltpu.sync_copy(x_hbm.at[seg], x_v)

        i32 = jnp.int32
        nchunks = _L // _LANES
        zv = jnp.zeros((_LANES,), i32)
        onev = jnp.full((_LANES,), 1, i32)

        def hist_body(i, carry):
            a0, a1, a2, a3 = carry
            lv = lab_v[pl.ds(i * _LANES, _LANES)]
            a0 = a0 + plsc.all_reduce_population_count(lv == 0)
            a1 = a1 + plsc.all_reduce_population_count(lv == 1)
            a2 = a2 + plsc.all_reduce_population_count(lv == 2)
            a3 = a3 + plsc.all_reduce_population_count(lv == 3)
            return a0, a1, a2, a3

        c0v, c1v, c2v, c3v = lax.fori_loop(0, nchunks, hist_body,
                                           (zv, zv, zv, zv))
        o1v = c0v
        o2v = c0v + c1v
        o3v = o2v + c2v
        o4v = o3v + c3v

        def part_body(i, carry):
            r0, r1, r2, r3, r4 = carry
            lv = lab_v[pl.ds(i * _LANES, _LANES)]
            xv = x_v[pl.ds(i * _LANES, _LANES)]
            outs = []
            for a, r in ((0, r0), (1, r1), (2, r2), (3, r3), (4, r4)):
                m = lv == a
                mi = jnp.where(m, onev, zv)
                pos = plsc.cumsum(mi)
                idx = r + pos - onev
                plsc.store_scatter(out_v, [idx], xv, mask=m)
                cnt = plsc.all_reduce_population_count(m)
                outs.append(r + cnt)
            return tuple(outs)

        init = (zv, o1v, o2v, o3v, o4v)
        lax.fori_loop(0, nchunks, part_body, init)

        io = lax.iota(i32, _LANES)
        cvec = (jnp.where(io == 0, c0v, zv) + jnp.where(io == 1, c1v, zv)
                + jnp.where(io == 2, c2v, zv) + jnp.where(io == 3, c3v, zv))
        cnt_v[...] = cvec
        pltpu.sync_copy(out_v, xs_hbm.at[seg])
        pltpu.sync_copy(cnt_v, cnt_hbm.at[seg])


@functools.cache
def _get_sc_partition():
    return pl.kernel(
        _sc_partition_kernel,
        out_type=(
            jax.ShapeDtypeStruct((_NSEG, _L), jnp.float32),
            jax.ShapeDtypeStruct((_NSEG, _LANES), jnp.int32),
        ),
        mesh=plsc.VectorSubcoreMesh(core_axis_name="c", subcore_axis_name="s"),
        compiler_params=pltpu.CompilerParams(needs_layout_passes=False),
        scratch_types=[
            pltpu.VMEM((_L,), jnp.int32),
            pltpu.VMEM((_L,), jnp.float32),
            pltpu.VMEM((_L,), jnp.float32),
            pltpu.VMEM((_LANES,), jnp.int32),
        ],
    )


def _tc_kernel(xs_ch_ref, xs_col_ref, cnt_ref, out_ref, sf_ref, si_ref,
               sacc_ref):
    s = pl.program_id(0)
    ib = pl.program_id(1)

    @pl.when(jnp.logical_and(s == 0, ib == 0))
    def _init():
        out_ref[...] = jnp.zeros((1, 1), jnp.float32)

    f32 = jnp.float32
    zero = jnp.float32(0.0)
    one = jnp.float32(1.0)

    @pl.when(ib == 0)
    def _per_segment():
        cnts = cnt_ref[0, 0]
        c0 = cnts[0]
        c1 = cnts[1]
        c2 = cnts[2]
        c3 = cnts[3]
        o1 = c0
        o2 = c0 + c1
        o3 = o2 + c2
        o4 = o3 + c3
        c1f = c1.astype(f32)
        c2f = c2.astype(f32)
        c3f = c3.astype(f32)
        n1f = o1.astype(f32)
        n2f = o2.astype(f32)
        n3f = o3.astype(f32)
        inc1 = jnp.logical_and(c1 > 0, o1 > 0).astype(f32)
        inc2 = jnp.logical_and(c2 > 0, o2 > 0).astype(f32)
        inc3 = jnp.logical_and(c3 > 0, o3 > 0).astype(f32)
        si_ref[0] = o1
        si_ref[1] = o2
        si_ref[2] = o3
        si_ref[3] = o4
        sf_ref[0] = inc1 / jnp.maximum(c1f * n1f, 1.0)
        sf_ref[1] = inc2 / jnp.maximum(c2f * n2f, 1.0)
        sf_ref[2] = inc3 / jnp.maximum(c3f * n3f, 1.0)
        sf_ref[3] = jnp.maximum(inc1 + inc2 + inc3, 1.0)
        sf_ref[4] = inc1 / jnp.maximum(c1f, 1.0)
        sf_ref[5] = inc2 / jnp.maximum(c2f, 1.0)
        sf_ref[6] = inc3 / jnp.maximum(c3f, 1.0)
        sf_ref[7] = jnp.max(xs_ch_ref[0])

    o1 = si_ref[0]
    o2 = si_ref[1]
    o3 = si_ref[2]
    o4 = si_ref[3]
    r1 = sf_ref[0]
    r2 = sf_ref[1]
    r3 = sf_ref[2]
    term_cnt = sf_ref[3]
    v1 = sf_ref[4]
    v2 = sf_ref[5]
    v3 = sf_ref[6]
    m = sf_ref[7]

    gi = ib * _BI + lax.broadcasted_iota(jnp.int32, (_BI, 1), 0)
    zi = jnp.zeros((_BI, 1), jnp.int32)
    zf = jnp.zeros((_BI, 1), f32)
    in1 = gi < o2
    in2 = gi < o3
    in3 = gi < o4
    in0 = gi < o1
    b_i = jnp.where(in0, zi,
                    jnp.where(in1, o1, jnp.where(in2, o2,
                                                 jnp.where(in3, o3, zi))))
    w_i = jnp.where(in0, zf,
                    jnp.where(in1, r1, jnp.where(in2, r2,
                                                 jnp.where(in3, r3, zf))))
    wl_i = jnp.where(in0, zf,
                     jnp.where(in1, v1, jnp.where(in2, v2,
                                                  jnp.where(in3, v3, zf))))

    xi = xs_col_ref[0]
    xim = xi - m
    ei = jnp.exp(xim)
    lin_b = jnp.sum(wl_i * xim)

    bmax = jnp.max(b_i)
    jio = lax.broadcasted_iota(jnp.int32, (1, _CH), 1)

    for c in range(_NCH):
        @pl.when(c * _CH < bmax)
        def _chunk(c=c):
            xch = xs_ch_ref[0, c]
            ec = jnp.exp(xch - m)
            mask = (jio + c * _CH) < b_i
            t = jnp.log(jnp.where(mask, ei + ec, one))
            sacc_ref[c] = jnp.sum(t * w_i)

        @pl.when(c * _CH >= bmax)
        def _skip(c=c):
            sacc_ref[c] = zero

    pair_log = sacc_ref[0] + sacc_ref[1] + sacc_ref[2] + sacc_ref[3]

    nseg = pl.num_programs(0)
    upd = -(lin_b - pair_log) / (term_cnt * nseg)
    out_ref[...] += jnp.broadcast_to(upd, (1, 1))


def _tc_loss(xs, cnt):
    xs_ch = xs.reshape(_NSEG, _NCH, 1, _CH)
    xs_col = xs.reshape(_NSEG, _L, 1)
    cnt3 = cnt.reshape(_NSEG, 1, _LANES)
    out = pl.pallas_call(
        _tc_kernel,
        grid=(_NSEG, _L // _BI),
        in_specs=[
            pl.BlockSpec((1, _NCH, 1, _CH), lambda s, ib: (s, 0, 0, 0)),
            pl.BlockSpec((1, _BI, 1), lambda s, ib: (s, ib, 0)),
            pl.BlockSpec((1, 1, _LANES), lambda s, ib: (s, 0, 0)),
        ],
        out_specs=pl.BlockSpec((1, 1), lambda s, ib: (0, 0)),
        out_shape=jax.ShapeDtypeStruct((1, 1), jnp.float32),
        scratch_shapes=[
            pltpu.SMEM((8,), jnp.float32),
            pltpu.SMEM((8,), jnp.int32),
            pltpu.SMEM((8,), jnp.float32),
        ],
    )(xs_ch, xs_col, cnt3)
    return out[0, 0]


@jax.jit
def _bpr_loss(logits, labels):
    x2d = logits.reshape(_NSEG, _L)
    lab2d = labels.reshape(_NSEG, _L)
    xs, cnt = _get_sc_partition()(x2d, lab2d)
    return _tc_loss(xs, cnt)


def kernel(s_num, logits, labels):
    return _bpr_loss(logits, labels)


# speedup vs baseline: 1.4700x; 1.4237x over previous
import functools

import jax
import jax.numpy as jnp
from jax import lax
from jax.experimental import pallas as pl
from jax.experimental.pallas import tpu as pltpu
from jax.experimental.pallas import tpu_sc as plsc


_NSEG = 16
_L = 2048
_BI = 512
_CH = 512
_SUB = 128
_NSUB = _CH // _SUB
_NCH = _L // _CH
_LANES = 16


def _sc_partition_kernel(x_hbm, lab_hbm, xs_hbm, cnt_hbm, lab_v, x_v, out_v,
                         cnt_v):
    core = lax.axis_index("c")
    sub = lax.axis_index("s")

    @pl.when(core == 0)
    def _body():
        seg = sub
        pltpu.sync_copy(lab_hbm.at[seg], lab_v)
        pltpu.sync_copy(x_hbm.at[seg], x_v)

        i32 = jnp.int32
        nchunks = _L // _LANES
        zv = jnp.zeros((_LANES,), i32)
        onev = jnp.full((_LANES,), 1, i32)

        def hist_body(i, carry):
            a0, a1, a2, a3 = carry
            lv = lab_v[pl.ds(i * _LANES, _LANES)]
            a0 = a0 + plsc.all_reduce_population_count(lv == 0)
            a1 = a1 + plsc.all_reduce_population_count(lv == 1)
            a2 = a2 + plsc.all_reduce_population_count(lv == 2)
            a3 = a3 + plsc.all_reduce_population_count(lv == 3)
            return a0, a1, a2, a3

        c0v, c1v, c2v, c3v = lax.fori_loop(0, nchunks, hist_body,
                                           (zv, zv, zv, zv))
        o1v = c0v
        o2v = c0v + c1v
        o3v = o2v + c2v
        o4v = o3v + c3v

        def part_body(i, carry):
            r0, r1, r2, r3, r4 = carry
            lv = lab_v[pl.ds(i * _LANES, _LANES)]
            xv = x_v[pl.ds(i * _LANES, _LANES)]
            outs = []
            for a, r in ((0, r0), (1, r1), (2, r2), (3, r3), (4, r4)):
                m = lv == a
                mi = jnp.where(m, onev, zv)
                pos = plsc.cumsum(mi)
                idx = r + pos - onev
                plsc.store_scatter(out_v, [idx], xv, mask=m)
                cnt = plsc.all_reduce_population_count(m)
                outs.append(r + cnt)
            return tuple(outs)

        init = (zv, o1v, o2v, o3v, o4v)
        lax.fori_loop(0, nchunks, part_body, init)

        io = lax.iota(i32, _LANES)
        cvec = (jnp.where(io == 0, c0v, zv) + jnp.where(io == 1, c1v, zv)
                + jnp.where(io == 2, c2v, zv) + jnp.where(io == 3, c3v, zv))
        cnt_v[...] = cvec
        pltpu.sync_copy(out_v, xs_hbm.at[seg])
        pltpu.sync_copy(cnt_v, cnt_hbm.at[seg])


@functools.cache
def _get_sc_partition():
    return pl.kernel(
        _sc_partition_kernel,
        out_type=(
            jax.ShapeDtypeStruct((_NSEG, _L), jnp.float32),
            jax.ShapeDtypeStruct((_NSEG, _LANES), jnp.int32),
        ),
        mesh=plsc.VectorSubcoreMesh(core_axis_name="c", subcore_axis_name="s"),
        compiler_params=pltpu.CompilerParams(needs_layout_passes=False),
        scratch_types=[
            pltpu.VMEM((_L,), jnp.int32),
            pltpu.VMEM((_L,), jnp.float32),
            pltpu.VMEM((_L,), jnp.float32),
            pltpu.VMEM((_LANES,), jnp.int32),
        ],
    )


def _tc_kernel(xs_ch_ref, xs_col_ref, cnt_ref, out_ref, sf_ref, si_ref,
               acc_ref):
    s = pl.program_id(0)
    ib = pl.program_id(1)

    @pl.when(jnp.logical_and(s == 0, ib == 0))
    def _init():
        out_ref[...] = jnp.zeros((1, 1), jnp.float32)

    f32 = jnp.float32
    zero = jnp.float32(0.0)
    one = jnp.float32(1.0)

    @pl.when(ib == 0)
    def _per_segment():
        cnts = cnt_ref[0, 0]
        c0 = cnts[0]
        c1 = cnts[1]
        c2 = cnts[2]
        c3 = cnts[3]
        o1 = c0
        o2 = c0 + c1
        o3 = o2 + c2
        o4 = o3 + c3
        c1f = c1.astype(f32)
        c2f = c2.astype(f32)
        c3f = c3.astype(f32)
        n1f = o1.astype(f32)
        n2f = o2.astype(f32)
        n3f = o3.astype(f32)
        inc1 = jnp.logical_and(c1 > 0, o1 > 0).astype(f32)
        inc2 = jnp.logical_and(c2 > 0, o2 > 0).astype(f32)
        inc3 = jnp.logical_and(c3 > 0, o3 > 0).astype(f32)
        si_ref[0] = o1
        si_ref[1] = o2
        si_ref[2] = o3
        si_ref[3] = o4
        sf_ref[0] = inc1 / jnp.maximum(c1f * n1f, 1.0)
        sf_ref[1] = inc2 / jnp.maximum(c2f * n2f, 1.0)
        sf_ref[2] = inc3 / jnp.maximum(c3f * n3f, 1.0)
        sf_ref[3] = jnp.maximum(inc1 + inc2 + inc3, 1.0)
        sf_ref[4] = inc1 / jnp.maximum(c1f, 1.0)
        sf_ref[5] = inc2 / jnp.maximum(c2f, 1.0)
        sf_ref[6] = inc3 / jnp.maximum(c3f, 1.0)
        sf_ref[7] = jnp.max(xs_ch_ref[0])

    o1 = si_ref[0]
    o2 = si_ref[1]
    o3 = si_ref[2]
    o4 = si_ref[3]
    r1 = sf_ref[0]
    r2 = sf_ref[1]
    r3 = sf_ref[2]
    term_cnt = sf_ref[3]
    v1 = sf_ref[4]
    v2 = sf_ref[5]
    v3 = sf_ref[6]
    m = sf_ref[7]

    gi = ib * _BI + lax.broadcasted_iota(jnp.int32, (_BI, 1), 0)
    zi = jnp.zeros((_BI, 1), jnp.int32)
    zf = jnp.zeros((_BI, 1), f32)
    in1 = gi < o2
    in2 = gi < o3
    in3 = gi < o4
    in0 = gi < o1
    b_i = jnp.where(in0, zi,
                    jnp.where(in1, o1, jnp.where(in2, o2,
                                                 jnp.where(in3, o3, zi))))
    w_i = jnp.where(in0, zf,
                    jnp.where(in1, r1, jnp.where(in2, r2,
                                                 jnp.where(in3, r3, zf))))
    wl_i = jnp.where(in0, zf,
                     jnp.where(in1, v1, jnp.where(in2, v2,
                                                  jnp.where(in3, v3, zf))))

    xi = xs_col_ref[0]
    xim = xi - m
    ei = jnp.exp(xim)
    lin_b = jnp.sum(wl_i * xim)

    bmax = jnp.max(b_i)
    jio = lax.broadcasted_iota(jnp.int32, (1, _SUB), 1)

    acc_ref[...] = jnp.zeros((_BI, _SUB), f32)
    for c in range(_NCH):
        @pl.when(c * _CH < bmax)
        def _chunk(c=c):
            xch = xs_ch_ref[0, c]
            ec = jnp.exp(xch - m)
            prod = jnp.full((_BI, _SUB), one, f32)
            for k in range(_NSUB):
                eck = ec[:, k * _SUB:(k + 1) * _SUB]
                mask = (jio + (c * _CH + k * _SUB)) < b_i
                prod = prod * jnp.where(mask, ei + eck, one)
            acc_ref[...] += jnp.log(prod)

    pair_log = jnp.sum(acc_ref[...] * w_i)

    nseg = pl.num_programs(0)
    upd = -(lin_b - pair_log) / (term_cnt * nseg)
    out_ref[...] += jnp.broadcast_to(upd, (1, 1))


def _tc_loss(xs, cnt):
    xs_ch = xs.reshape(_NSEG, _NCH, 1, _CH)
    xs_col = xs.reshape(_NSEG, _L, 1)
    cnt3 = cnt.reshape(_NSEG, 1, _LANES)
    out = pl.pallas_call(
        _tc_kernel,
        grid=(_NSEG, _L // _BI),
        in_specs=[
            pl.BlockSpec((1, _NCH, 1, _CH), lambda s, ib: (s, 0, 0, 0)),
            pl.BlockSpec((1, _BI, 1), lambda s, ib: (s, ib, 0)),
            pl.BlockSpec((1, 1, _LANES), lambda s, ib: (s, 0, 0)),
        ],
        out_specs=pl.BlockSpec((1, 1), lambda s, ib: (0, 0)),
        out_shape=jax.ShapeDtypeStruct((1, 1), jnp.float32),
        scratch_shapes=[
            pltpu.SMEM((8,), jnp.float32),
            pltpu.SMEM((8,), jnp.int32),
            pltpu.VMEM((_BI, _SUB), jnp.float32),
        ],
    )(xs_ch, xs_col, cnt3)
    return out[0, 0]


@jax.jit
def _bpr_loss(logits, labels):
    x2d = logits.reshape(_NSEG, _L)
    lab2d = labels.reshape(_NSEG, _L)
    xs, cnt = _get_sc_partition()(x2d, lab2d)
    return _tc_loss(xs, cnt)


def kernel(s_num, logits, labels):
    return _bpr_loss(logits, labels)
